# Initial kernel scaffold; baseline (speedup 1.0000x reference)
import jax
import jax.numpy as jnp
from jax.experimental import pallas as pl


def kernel(x, edge_index, batch, train_mask):
    raise NotImplementedError("write your pallas kernel here")


# speedup vs baseline: 3.1360x; 3.1360x over previous
import functools

import jax
import jax.numpy as jnp
from jax import lax
from jax.experimental import pallas as pl
from jax.experimental.pallas import tpu as pltpu
from jax.experimental.pallas import tpu_sc as plsc

NUM_SEGMENTS = 64
NUM_ROWS = 10000
NUM_COLS = 128
CHUNK = 80
NUM_CHUNKS = NUM_ROWS // CHUNK
NUM_WORKERS = 32
MAX_CHUNKS_PER_WORKER = -(-NUM_CHUNKS // NUM_WORKERS)


def _sc_partials(x, batch):
    mesh = plsc.VectorSubcoreMesh(core_axis_name="c", subcore_axis_name="s")

    @functools.partial(
        pl.kernel,
        out_type=jax.ShapeDtypeStruct((2, NUM_SEGMENTS, NUM_COLS), jnp.float32),
        mesh=mesh,
        scratch_types=[
            pltpu.VMEM((CHUNK,), jnp.int32),
            pltpu.VMEM((CHUNK, NUM_COLS), jnp.float32),
            pltpu.VMEM((NUM_SEGMENTS // 16, NUM_COLS), jnp.float32),
            pltpu.VMEM_SHARED((NUM_SEGMENTS, NUM_COLS), jnp.float32),
        ],
    )
    def k(x_hbm, batch_hbm, part_hbm, idx_v, rows_v, zero_v, acc_sh):
        cid = lax.axis_index("c")
        sid = lax.axis_index("s")
        wid = sid * 2 + cid

        zrows = NUM_SEGMENTS // 16
        for r in range(zrows):
            for cb in range(NUM_COLS // 16):
                zero_v[r, pl.ds(cb * 16, 16)] = jnp.zeros((16,), jnp.float32)
        pltpu.sync_copy(zero_v, acc_sh.at[pl.ds(sid * zrows, zrows)])
        plsc.subcore_barrier()

        for k_ in range(MAX_CHUNKS_PER_WORKER):
            c = wid + NUM_WORKERS * k_

            @pl.when(c < NUM_CHUNKS)
            def _():
                base = c * CHUNK
                pltpu.sync_copy(batch_hbm.at[pl.ds(base, CHUNK)], idx_v)
                pltpu.sync_copy(x_hbm.at[pl.ds(base, CHUNK)], rows_v)
                pltpu.sync_copy(rows_v, acc_sh.at[idx_v], add=True)

        plsc.subcore_barrier()

        @pl.when(sid == 0)
        def _():
            pltpu.sync_copy(acc_sh, part_hbm.at[cid])

    return k(x, batch)


def _combine(p_ref, o1_ref, o2_ref):
    s = p_ref[0] + p_ref[1]
    o1_ref[...] = s
    o2_ref[...] = s


def kernel(x, edge_index, batch, train_mask):
    del edge_index, train_mask
    partials = _sc_partials(x, batch)
    out_sds = jax.ShapeDtypeStruct((NUM_SEGMENTS, NUM_COLS), jnp.float32)
    m1, m2 = pl.pallas_call(_combine, out_shape=(out_sds, out_sds))(partials)
    return (m1, m2)


# speedup vs baseline: 3.4852x; 1.1114x over previous
import functools

import jax
import jax.numpy as jnp
from jax import lax
from jax.experimental import pallas as pl
from jax.experimental.pallas import tpu as pltpu
from jax.experimental.pallas import tpu_sc as plsc

NUM_SEGMENTS = 64
NUM_ROWS = 10000
NUM_COLS = 128
CHUNK = 80
NUM_CHUNKS = NUM_ROWS // CHUNK
NUM_WORKERS = 32
MAX_CHUNKS_PER_WORKER = -(-NUM_CHUNKS // NUM_WORKERS)


def _sc_partials(x, batch):
    mesh = plsc.VectorSubcoreMesh(core_axis_name="c", subcore_axis_name="s")

    @functools.partial(
        pl.kernel,
        out_type=jax.ShapeDtypeStruct((2, NUM_SEGMENTS, NUM_COLS), jnp.float32),
        mesh=mesh,
        scratch_types=[
            pltpu.VMEM((MAX_CHUNKS_PER_WORKER, CHUNK), jnp.int32),
            pltpu.VMEM((2, CHUNK, NUM_COLS), jnp.float32),
            pltpu.VMEM((NUM_SEGMENTS // 16, NUM_COLS), jnp.float32),
            pltpu.VMEM_SHARED((NUM_SEGMENTS, NUM_COLS), jnp.float32),
            pltpu.SemaphoreType.DMA,
            pltpu.SemaphoreType.DMA,
        ],
    )
    def k(x_hbm, batch_hbm, part_hbm, idx_v, rows_v, zero_v, acc_sh, sem_a, sem_b):
        cid = lax.axis_index("c")
        sid = lax.axis_index("s")
        wid = sid * 2 + cid
        sems = [sem_a, sem_b]

        def start(k_, c):
            base = c * CHUNK
            sem = sems[k_ % 2]
            pltpu.async_copy(batch_hbm.at[pl.ds(base, CHUNK)], idx_v.at[k_], sem)
            pltpu.async_copy(x_hbm.at[pl.ds(base, CHUNK)], rows_v.at[k_ % 2], sem)

        def wait(k_, c):
            base = c * CHUNK
            sem = sems[k_ % 2]
            pltpu.make_async_copy(
                batch_hbm.at[pl.ds(base, CHUNK)], idx_v.at[k_], sem).wait()
            pltpu.make_async_copy(
                x_hbm.at[pl.ds(base, CHUNK)], rows_v.at[k_ % 2], sem).wait()

        start(0, wid)

        zrows = NUM_SEGMENTS // 16
        for r in range(zrows):
            for cb in range(NUM_COLS // 16):
                zero_v[r, pl.ds(cb * 16, 16)] = jnp.zeros((16,), jnp.float32)
        pltpu.sync_copy(zero_v, acc_sh.at[pl.ds(sid * zrows, zrows)])
        plsc.subcore_barrier()

        for k_ in range(MAX_CHUNKS_PER_WORKER):
            c = wid + NUM_WORKERS * k_

            @pl.when(c < NUM_CHUNKS)
            def _():
                wait(k_, c)
                if k_ + 1 < MAX_CHUNKS_PER_WORKER:
                    @pl.when(c + NUM_WORKERS < NUM_CHUNKS)
                    def _():
                        start(k_ + 1, c + NUM_WORKERS)
                pltpu.sync_copy(rows_v.at[k_ % 2], acc_sh.at[idx_v.at[k_]], add=True)

        plsc.subcore_barrier()

        @pl.when(sid == 0)
        def _():
            pltpu.sync_copy(acc_sh, part_hbm.at[cid])

    return k(x, batch)


def _combine(p_ref, o1_ref, o2_ref):
    s = p_ref[0] + p_ref[1]
    o1_ref[...] = s
    o2_ref[...] = s


def kernel(x, edge_index, batch, train_mask):
    del edge_index, train_mask
    partials = _sc_partials(x, batch)
    out_sds = jax.ShapeDtypeStruct((NUM_SEGMENTS, NUM_COLS), jnp.float32)
    m1, m2 = pl.pallas_call(_combine, out_shape=(out_sds, out_sds))(partials)
    return (m1, m2)


# speedup vs baseline: 3.5778x; 1.0266x over previous
import functools

import jax
import jax.numpy as jnp
from jax import lax
from jax.experimental import pallas as pl
from jax.experimental.pallas import tpu as pltpu
from jax.experimental.pallas import tpu_sc as plsc

NUM_SEGMENTS = 64
NUM_ROWS = 10000
NUM_COLS = 128
CHUNK = 80
NUM_CHUNKS = NUM_ROWS // CHUNK
NUM_WORKERS = 32
MAX_CHUNKS_PER_WORKER = -(-NUM_CHUNKS // NUM_WORKERS)


def _sc_partials(x, batch):
    mesh = plsc.VectorSubcoreMesh(core_axis_name="c", subcore_axis_name="s")

    @functools.partial(
        pl.kernel,
        out_type=jax.ShapeDtypeStruct((2, NUM_SEGMENTS, NUM_COLS), jnp.float32),
        mesh=mesh,
        scratch_types=[
            pltpu.VMEM((MAX_CHUNKS_PER_WORKER, CHUNK), jnp.int32),
            pltpu.VMEM((MAX_CHUNKS_PER_WORKER, CHUNK, NUM_COLS), jnp.float32),
            pltpu.VMEM((NUM_SEGMENTS // 16, NUM_COLS), jnp.float32),
            pltpu.VMEM_SHARED((NUM_SEGMENTS, NUM_COLS), jnp.float32),
            [pltpu.SemaphoreType.DMA] * MAX_CHUNKS_PER_WORKER,
        ],
    )
    def k(x_hbm, batch_hbm, part_hbm, idx_v, rows_v, zero_v, acc_sh, sems):
        cid = lax.axis_index("c")
        sid = lax.axis_index("s")
        wid = sid * 2 + cid

        def start(k_, c):
            base = c * CHUNK
            pltpu.async_copy(batch_hbm.at[pl.ds(base, CHUNK)], idx_v.at[k_], sems[k_])
            pltpu.async_copy(x_hbm.at[pl.ds(base, CHUNK)], rows_v.at[k_], sems[k_])

        def wait(k_, c):
            base = c * CHUNK
            pltpu.make_async_copy(
                batch_hbm.at[pl.ds(base, CHUNK)], idx_v.at[k_], sems[k_]).wait()
            pltpu.make_async_copy(
                x_hbm.at[pl.ds(base, CHUNK)], rows_v.at[k_], sems[k_]).wait()

        for k_ in range(MAX_CHUNKS_PER_WORKER):
            c = wid + NUM_WORKERS * k_

            @pl.when(c < NUM_CHUNKS)
            def _():
                start(k_, c)

        zrows = NUM_SEGMENTS // 16
        for r in range(zrows):
            for cb in range(NUM_COLS // 16):
                zero_v[r, pl.ds(cb * 16, 16)] = jnp.zeros((16,), jnp.float32)
        pltpu.sync_copy(zero_v, acc_sh.at[pl.ds(sid * zrows, zrows)])
        plsc.subcore_barrier()

        for k_ in range(MAX_CHUNKS_PER_WORKER):
            c = wid + NUM_WORKERS * k_

            @pl.when(c < NUM_CHUNKS)
            def _():
                wait(k_, c)
                pltpu.sync_copy(rows_v.at[k_], acc_sh.at[idx_v.at[k_]], add=True)

        plsc.subcore_barrier()

        @pl.when(sid == 0)
        def _():
            pltpu.sync_copy(acc_sh, part_hbm.at[cid])

    return k(x, batch)


def _combine(p_ref, o1_ref, o2_ref):
    s = p_ref[0] + p_ref[1]
    o1_ref[...] = s
    o2_ref[...] = s


def kernel(x, edge_index, batch, train_mask):
    del edge_index, train_mask
    partials = _sc_partials(x, batch)
    out_sds = jax.ShapeDtypeStruct((NUM_SEGMENTS, NUM_COLS), jnp.float32)
    m1, m2 = pl.pallas_call(_combine, out_shape=(out_sds, out_sds))(partials)
    return (m1, m2)
